# baseline (speedup 1.0000x reference)
import functools

import jax
import jax.numpy as jnp
from jax import lax
from jax.experimental import pallas as pl
from jax.experimental.pallas import tpu as pltpu
from jax.experimental.pallas import tpu_sc as plsc

_NUM_EMBEDDINGS = 1000
_SCALE = float(_NUM_EMBEDDINGS - 1)
_MAGIC = 1.5 * 2.0**23

_NC = 2
_NS = 16
_NW = _NC * _NS
_L = 16

_CHUNK = 25_600


def _quantize_body(x_hbm, out_hbm, in_v, out_v):
    n = x_hbm.shape[0]
    per_w = n // _NW
    nchunk = per_w // _CHUNK
    wid = lax.axis_index("c") * _NS + lax.axis_index("s")
    base = wid * per_w

    def chunk_step(ci, carry):
        off = base + ci * _CHUNK
        pltpu.sync_copy(x_hbm.at[pl.ds(off, _CHUNK)], in_v)

        def vec_step(vi, c):
            v = in_v[pl.ds(vi * _L, _L)]
            y = v * _SCALE
            y = (y + _MAGIC) - _MAGIC
            y = jnp.minimum(jnp.maximum(y, 0.0), _SCALE)
            out_v[pl.ds(vi * _L, _L)] = y.astype(jnp.int32)
            return c

        lax.fori_loop(0, _CHUNK // _L, vec_step, 0)
        pltpu.sync_copy(out_v, out_hbm.at[pl.ds(off, _CHUNK)])
        return carry

    lax.fori_loop(0, nchunk, chunk_step, 0)


def kernel(x):
    b, t, _ = x.shape
    n = b * t
    assert n % (_NW * _CHUNK) == 0
    xf = x.reshape(n)
    f = pl.kernel(
        _quantize_body,
        out_type=jax.ShapeDtypeStruct((n,), jnp.int32),
        mesh=plsc.VectorSubcoreMesh(core_axis_name="c", subcore_axis_name="s"),
        scratch_types=[
            pltpu.VMEM((_CHUNK,), jnp.float32),
            pltpu.VMEM((_CHUNK,), jnp.int32),
        ],
    )
    out = f(xf)
    return out.reshape(b, t).astype(jnp.int64)


# speedup vs baseline: 1.1963x; 1.1963x over previous
import jax
import jax.numpy as jnp
from jax import lax
from jax.experimental import pallas as pl
from jax.experimental.pallas import tpu as pltpu
from jax.experimental.pallas import tpu_sc as plsc

_NUM_EMBEDDINGS = 1000
_SCALE = float(_NUM_EMBEDDINGS - 1)
_MAGIC = 1.5 * 2.0**23
_MAGIC_BITS = 0x4B400000
_BIASED_MAX = _MAGIC + _SCALE

_NC = 2
_NS = 16
_NW = _NC * _NS
_L = 16

_CHUNK = 25_600
_NBUF = 2


def _quantize_body(x_hbm, out_hbm, in_bufs, out_bufs, in_sems, out_sems):
    n = x_hbm.shape[0]
    per_w = n // _NW
    nchunk = per_w // _CHUNK
    wid = lax.axis_index("c") * _NS + lax.axis_index("s")
    base = wid * per_w

    def start_load(ci):
        off = base + ci * _CHUNK
        return pltpu.async_copy(
            x_hbm.at[pl.ds(off, _CHUNK)], in_bufs[ci % _NBUF], in_sems[ci % _NBUF]
        )

    def start_store(ci):
        off = base + ci * _CHUNK
        return pltpu.async_copy(
            out_bufs[ci % _NBUF], out_hbm.at[pl.ds(off, _CHUNK)], out_sems[ci % _NBUF]
        )

    loads = {0: start_load(0)}
    stores = {}
    for ci in range(nchunk):
        if ci + 1 < nchunk:
            loads[ci + 1] = start_load(ci + 1)
        loads.pop(ci).wait()
        if ci - _NBUF in stores:
            stores.pop(ci - _NBUF).wait()
        in_b = in_bufs[ci % _NBUF]
        out_b = out_bufs[ci % _NBUF]

        @plsc.parallel_loop(0, _CHUNK, step=_L, unroll=8)
        def _(i):
            t = in_b[pl.ds(i, _L)] * _SCALE + _MAGIC
            t = jnp.minimum(jnp.maximum(t, _MAGIC), _BIASED_MAX)
            out_b[pl.ds(i, _L)] = (t - _MAGIC).astype(jnp.int32)

        stores[ci] = start_store(ci)
    for d in stores.values():
        d.wait()


def kernel(x):
    b, t, _ = x.shape
    n = b * t
    assert n % (_NW * _CHUNK) == 0
    xf = x.reshape(n)
    f = pl.kernel(
        _quantize_body,
        out_type=jax.ShapeDtypeStruct((n,), jnp.int32),
        mesh=plsc.VectorSubcoreMesh(core_axis_name="c", subcore_axis_name="s"),
        scratch_types=[
            [pltpu.VMEM((_CHUNK,), jnp.float32) for _ in range(_NBUF)],
            [pltpu.VMEM((_CHUNK,), jnp.int32) for _ in range(_NBUF)],
            [pltpu.SemaphoreType.DMA for _ in range(_NBUF)],
            [pltpu.SemaphoreType.DMA for _ in range(_NBUF)],
        ],
    )
    out = f(xf)
    return out.reshape(b, t).astype(jnp.int64)


# speedup vs baseline: 1.7250x; 1.4419x over previous
import jax
import jax.numpy as jnp
from jax import lax
from jax.experimental import pallas as pl
from jax.experimental.pallas import tpu as pltpu
from jax.experimental.pallas import tpu_sc as plsc

_NUM_EMBEDDINGS = 1000
_SCALE = float(_NUM_EMBEDDINGS - 1)
_MAGIC = 1.5 * 2.0**23
_BIASED_MAX = _MAGIC + _SCALE

_NC = 2
_NS = 16
_NW = _NC * _NS
_L = 16
_BBLK = 128
_NBUF = 2


def _quantize_body(x_hbm, out_hbm, in_bufs, out_bufs, in_sems, out_sems):
    t_dim, b_dim = x_hbm.shape
    tb_dim = t_dim // 8
    per_w = b_dim // _BBLK // _NW
    wid = lax.axis_index("c") * _NS + lax.axis_index("s")

    def start_load(ci):
        bb = wid * per_w + ci
        return pltpu.async_copy(
            x_hbm.at[:, pl.ds(bb * _BBLK, _BBLK)], in_bufs[ci % _NBUF],
            in_sems[ci % _NBUF],
        )

    def start_store(ci):
        bb = wid * per_w + ci
        return pltpu.async_copy(
            out_bufs[ci % _NBUF], out_hbm.at[:, bb, :], out_sems[ci % _NBUF]
        )

    loads = {0: start_load(0)}
    stores = {}
    for ci in range(per_w):
        if ci + 1 < per_w:
            loads[ci + 1] = start_load(ci + 1)
        loads.pop(ci).wait()
        if ci - _NBUF in stores:
            stores.pop(ci - _NBUF).wait()
        in_b = in_bufs[ci % _NBUF]
        out_b = out_bufs[ci % _NBUF]

        @plsc.parallel_loop(0, t_dim)
        def _(row):
            tb = row // 8
            col0 = (row - tb * 8) * _BBLK
            for k in range(_BBLK // _L):
                v = in_b[row, pl.ds(k * _L, _L)]
                y = v * _SCALE + _MAGIC
                y = jnp.minimum(jnp.maximum(y, _MAGIC), _BIASED_MAX)
                out_b[tb, pl.ds(col0 + k * _L, _L)] = (y - _MAGIC).astype(
                    jnp.int32
                )

        stores[ci] = start_store(ci)
    for d in stores.values():
        d.wait()


def kernel(x):
    b, t, _ = x.shape
    assert t % 8 == 0 and b % (_BBLK * _NW) == 0
    xt = jnp.swapaxes(x.squeeze(-1), 0, 1)
    f = pl.kernel(
        _quantize_body,
        out_type=jax.ShapeDtypeStruct((t // 8, b // _BBLK, 8 * _BBLK), jnp.int32),
        mesh=plsc.VectorSubcoreMesh(core_axis_name="c", subcore_axis_name="s"),
        scratch_types=[
            [pltpu.VMEM((t, _BBLK), jnp.float32) for _ in range(_NBUF)],
            [pltpu.VMEM((t // 8, 8 * _BBLK), jnp.int32) for _ in range(_NBUF)],
            [pltpu.SemaphoreType.DMA for _ in range(_NBUF)],
            [pltpu.SemaphoreType.DMA for _ in range(_NBUF)],
        ],
    )
    o3 = f(xt)
    out = o3.reshape(t // 8, b // _BBLK, 8, _BBLK).transpose(1, 3, 0, 2)
    return out.reshape(b, t).astype(jnp.int64)


# speedup vs baseline: 2.0004x; 1.1597x over previous
import jax
import jax.numpy as jnp
from jax import lax
from jax.experimental import pallas as pl
from jax.experimental.pallas import tpu as pltpu
from jax.experimental.pallas import tpu_sc as plsc

_NUM_EMBEDDINGS = 1000
_SCALE = float(_NUM_EMBEDDINGS - 1)
_MAGIC = 1.5 * 2.0**23
_BIASED_MAX = _MAGIC + _SCALE

_NC = 2
_NS = 16
_NW = _NC * _NS
_L = 16

_TI = 8
_BH = 8
_NBUF = 2


def _quantize_body(x_hbm, out_hbm, in_bufs, out_bufs, in_sems, out_sems):
    t_dim = x_hbm.shape[0]
    nbh = x_hbm.shape[1]
    n_units = (t_dim // _TI) * (nbh // _BH)
    ring_units = n_units // _NW * _NW
    per_w = ring_units // _NW
    n_tail = n_units - ring_units
    wid = lax.axis_index("c") * _NS + lax.axis_index("s")

    def unit_slices(u):
        tb = u // (nbh // _BH)
        bh0 = (u - tb * (nbh // _BH)) * _BH
        src = x_hbm.at[pl.ds(tb * _TI, _TI), pl.ds(bh0, _BH), :]
        dst = out_hbm.at[tb, pl.ds(bh0, _BH), :, :]
        return src, dst

    def compute(b):
        in_b, out_b = in_bufs[b], out_bufs[b]

        @plsc.parallel_loop(0, _TI * _BH)
        def _(p):
            ti = p // _BH
            bh = p - ti * _BH
            for k in range(128 // _L):
                v = in_b[ti, bh, pl.ds(k * _L, _L)]
                y = v * _SCALE + _MAGIC
                y = jnp.minimum(jnp.maximum(y, _MAGIC), _BIASED_MAX)
                out_b[bh, ti, pl.ds(k * _L, _L)] = (y - _MAGIC).astype(
                    jnp.int32
                )

    for b in range(_NBUF):
        src, _ = unit_slices(wid + _NW * b)
        pltpu.async_copy(src, in_bufs[b], in_sems[b])

    @pl.loop(0, per_w, step=_NBUF)
    def _(m):
        for b in range(_NBUF):
            j = m + b
            u = wid + _NW * j
            src, dst = unit_slices(u)
            pltpu.make_async_copy(src, in_bufs[b], in_sems[b]).wait()

            @pl.when(j + _NBUF < per_w)
            def _():
                src2, _ = unit_slices(u + _NW * _NBUF)
                pltpu.async_copy(src2, in_bufs[b], in_sems[b])

            @pl.when(j >= _NBUF)
            def _():
                _, dprev = unit_slices(u - _NW * _NBUF)
                pltpu.make_async_copy(out_bufs[b], dprev, out_sems[b]).wait()

            compute(b)
            pltpu.async_copy(out_bufs[b], dst, out_sems[b])

    for b in range(_NBUF):
        _, dst = unit_slices(wid + _NW * (per_w - _NBUF + b))
        pltpu.make_async_copy(out_bufs[b], dst, out_sems[b]).wait()

    @pl.when(wid < n_tail)
    def _():
        src, dst = unit_slices(ring_units + wid)
        pltpu.async_copy(src, in_bufs[0], in_sems[0]).wait()
        compute(0)
        pltpu.async_copy(out_bufs[0], dst, out_sems[0]).wait()


def kernel(x):
    b, t, _ = x.shape
    assert t % _TI == 0 and b % (128 * _BH) == 0
    xt = jnp.swapaxes(x.squeeze(-1), 0, 1)
    x3 = xt.reshape(t, b // 128, 128)
    f = pl.kernel(
        _quantize_body,
        out_type=jax.ShapeDtypeStruct((t // _TI, b // 128, _TI, 128), jnp.int32),
        mesh=plsc.VectorSubcoreMesh(core_axis_name="c", subcore_axis_name="s"),
        scratch_types=[
            [pltpu.VMEM((_TI, _BH, 128), jnp.float32) for _ in range(_NBUF)],
            [pltpu.VMEM((_BH, _TI, 128), jnp.int32) for _ in range(_NBUF)],
            [pltpu.SemaphoreType.DMA for _ in range(_NBUF)],
            [pltpu.SemaphoreType.DMA for _ in range(_NBUF)],
        ],
    )
    o4 = f(x3)
    out = o4.transpose(1, 3, 0, 2).reshape(b, t)
    return out.astype(jnp.int64)
